# baseline (speedup 1.0000x reference)
import numpy as np
import jax
import jax.numpy as jnp
from jax.experimental import pallas as pl
from jax.experimental.pallas import tpu as pltpu

_ALL_ANCHORS = np.array(
    [[10, 13], [16, 30], [33, 23], [30, 61], [62, 45], [59, 119],
     [116, 90], [156, 198], [373, 326]], dtype=np.float32)
_ANCHOR_IDXS = np.array([0, 1, 2])
_NCLS = 80
_A = 3
_CH = 5 + _NCLS
_C_IN = 128
_N = 64
_HW = _N * _N
_T = 4
_HW_T = _HW // _T
_STRIDE = 8.0

_k = np.arange(_CH)
_M_SIG = np.where(_k < 2, _STRIDE, np.where(_k < 4, 0.0, 1.0)).astype(np.float32)
_M_X = np.where(_k == 0, _STRIDE, 0.0).astype(np.float32)
_M_Y = np.where(_k == 1, _STRIDE, 0.0).astype(np.float32)
_M_EXP = np.zeros((_A, 1, _CH), np.float32)
_anch = _ALL_ANCHORS[_ANCHOR_IDXS]
for _a in range(_A):
    _M_EXP[_a, 0, 2] = _anch[_a, 0]
    _M_EXP[_a, 0, 3] = _anch[_a, 1]


def _yolo_kernel(x_ref, w_ref, b_ref, msig_ref, mexp_ref, mx_ref, my_ref,
                 out_ref):
    j = pl.program_id(1)
    t = j // _A
    xb = x_ref[0]
    w = w_ref[0]
    y = jax.lax.dot_general(
        xb, w, (((0,), (1,)), ((), ())),
        preferred_element_type=jnp.float32)
    y = y + b_ref[0]
    sig = jax.nn.sigmoid(y)
    mexp = mexp_ref[0]
    ex = jnp.exp(jnp.where(mexp != 0.0, y, 0.0)) * mexp
    hw = t * _HW_T + jax.lax.broadcasted_iota(jnp.int32, (_HW_T, 1), 0)
    wcol = (hw & (_N - 1)).astype(jnp.float32)
    hcol = (hw >> 6).astype(jnp.float32)
    res = sig * msig_ref[...] + ex + wcol * mx_ref[...] + hcol * my_ref[...]
    out_ref[0] = res


def kernel(x, conv_w, conv_b):
    B = x.shape[0]
    xf = x.reshape(B, _C_IN, _HW)
    w = conv_w[:, :, 0, 0].reshape(_A, _CH, _C_IN)
    b = conv_b.reshape(_A, 1, _CH)
    msig = jnp.asarray(_M_SIG).reshape(1, _CH)
    mexp = jnp.asarray(_M_EXP)
    mx = jnp.asarray(_M_X).reshape(1, _CH)
    my = jnp.asarray(_M_Y).reshape(1, _CH)

    out = pl.pallas_call(
        _yolo_kernel,
        grid=(B, _T * _A),
        in_specs=[
            pl.BlockSpec((1, _C_IN, _HW_T), lambda i, j: (i, 0, j // _A)),
            pl.BlockSpec((1, _CH, _C_IN), lambda i, j: (j % _A, 0, 0)),
            pl.BlockSpec((1, 1, _CH), lambda i, j: (j % _A, 0, 0)),
            pl.BlockSpec((1, _CH), lambda i, j: (0, 0)),
            pl.BlockSpec((1, 1, _CH), lambda i, j: (j % _A, 0, 0)),
            pl.BlockSpec((1, _CH), lambda i, j: (0, 0)),
            pl.BlockSpec((1, _CH), lambda i, j: (0, 0)),
        ],
        out_specs=pl.BlockSpec(
            (1, _HW_T, _CH), lambda i, j: (i, (j % _A) * _T + j // _A, 0)),
        out_shape=jax.ShapeDtypeStruct((B, _A * _HW, _CH), jnp.float32),
        compiler_params=pltpu.CompilerParams(
            dimension_semantics=("arbitrary", "arbitrary")),
    )(xf, w, b, msig, mexp, mx, my)
    return out


# speedup vs baseline: 1.6551x; 1.6551x over previous
import numpy as np
import jax
import jax.numpy as jnp
from jax.experimental import pallas as pl
from jax.experimental.pallas import tpu as pltpu

_ALL_ANCHORS = np.array(
    [[10, 13], [16, 30], [33, 23], [30, 61], [62, 45], [59, 119],
     [116, 90], [156, 198], [373, 326]], dtype=np.float32)
_ANCHOR_IDXS = np.array([0, 1, 2])
_NCLS = 80
_A = 3
_CH = 5 + _NCLS
_C_OUT = _A * _CH
_C_IN = 128
_N = 64
_HW = _N * _N
_STRIDE = 8.0

_o = np.arange(_C_OUT)
_k = _o % _CH
_M_SIG = np.where(_k < 2, _STRIDE, np.where(_k < 4, 0.0, 1.0)).astype(np.float32)
_M_EXP = np.zeros(_C_OUT, np.float32)
_anch = _ALL_ANCHORS[_ANCHOR_IDXS]
for _a in range(_A):
    _M_EXP[_a * _CH + 2] = _anch[_a, 0]
    _M_EXP[_a * _CH + 3] = _anch[_a, 1]
_M_X = np.where(_k == 0, _STRIDE, 0.0).astype(np.float32)
_M_Y = np.where(_k == 1, _STRIDE, 0.0).astype(np.float32)


def _yolo_kernel(x_ref, w_ref, b_ref, msig_ref, mexp_ref, mx_ref, my_ref,
                 out_ref):
    xb = x_ref[0]
    w = w_ref[...]
    y = jax.lax.dot_general(
        xb, w, (((0,), (1,)), ((), ())),
        preferred_element_type=jnp.float32)
    y = y + b_ref[...]
    sig = jax.nn.sigmoid(y)
    mexp = mexp_ref[...]
    ex = jnp.exp(jnp.where(mexp != 0.0, y, 0.0)) * mexp
    row = jax.lax.broadcasted_iota(jnp.int32, (_HW, 1), 0)
    wcol = (row & (_N - 1)).astype(jnp.float32)
    hcol = (row >> 6).astype(jnp.float32)
    res = sig * msig_ref[...] + ex + wcol * mx_ref[...] + hcol * my_ref[...]
    for a in range(_A):
        out_ref[0, pl.ds(_HW * a, _HW), :] = res[:, _CH * a:_CH * (a + 1)]


def kernel(x, conv_w, conv_b):
    B = x.shape[0]
    xf = x.reshape(B, _C_IN, _HW)
    w = conv_w[:, :, 0, 0]
    b = conv_b.reshape(1, _C_OUT)
    msig = jnp.asarray(_M_SIG).reshape(1, _C_OUT)
    mexp = jnp.asarray(_M_EXP).reshape(1, _C_OUT)
    mx = jnp.asarray(_M_X).reshape(1, _C_OUT)
    my = jnp.asarray(_M_Y).reshape(1, _C_OUT)

    out = pl.pallas_call(
        _yolo_kernel,
        grid=(B,),
        in_specs=[
            pl.BlockSpec((1, _C_IN, _HW), lambda i: (i, 0, 0)),
            pl.BlockSpec((_C_OUT, _C_IN), lambda i: (0, 0)),
            pl.BlockSpec((1, _C_OUT), lambda i: (0, 0)),
            pl.BlockSpec((1, _C_OUT), lambda i: (0, 0)),
            pl.BlockSpec((1, _C_OUT), lambda i: (0, 0)),
            pl.BlockSpec((1, _C_OUT), lambda i: (0, 0)),
            pl.BlockSpec((1, _C_OUT), lambda i: (0, 0)),
        ],
        out_specs=pl.BlockSpec((1, _A * _HW, _CH), lambda i: (i, 0, 0)),
        out_shape=jax.ShapeDtypeStruct((B, _A * _HW, _CH), jnp.float32),
        compiler_params=pltpu.CompilerParams(
            dimension_semantics=("arbitrary",)),
    )(xf, w, b, msig, mexp, mx, my)
    return out


# speedup vs baseline: 6.3531x; 3.8384x over previous
import numpy as np
import jax
import jax.numpy as jnp
from jax.experimental import pallas as pl
from jax.experimental.pallas import tpu as pltpu

_ALL_ANCHORS = np.array(
    [[10, 13], [16, 30], [33, 23], [30, 61], [62, 45], [59, 119],
     [116, 90], [156, 198], [373, 326]], dtype=np.float32)
_ANCHOR_IDXS = np.array([0, 1, 2])
_NCLS = 80
_A = 3
_CH = 5 + _NCLS
_C_OUT = _A * _CH
_C_IN = 128
_N = 64
_HW = _N * _N
_STRIDE = 8.0

_o = np.arange(_C_OUT)
_k = _o % _CH
_M_SIG = np.where(_k < 2, _STRIDE, 1.0).astype(np.float32)
_anch = _ALL_ANCHORS[_ANCHOR_IDXS]

_hw = np.arange(_HW)
_OFF = np.zeros((_C_OUT, _HW), np.float32)
_OFF[_k == 0, :] = (_STRIDE * (_hw % _N))[None, :]
_OFF[_k == 1, :] = (_STRIDE * (_hw // _N))[None, :]


def _yolo_kernel(x_ref, w_ref, b_ref, msig_ref, off_ref, out_ref,
                 slab_ref, sem):
    i = pl.program_id(0)
    nb = pl.num_programs(0)
    s = i % 2

    @pl.when(i >= 2)
    def _():
        pltpu.make_async_copy(
            slab_ref.at[s], out_ref.at[:, i - 2, :], sem.at[s]).wait()

    xb = x_ref[0]
    w = w_ref[...]
    y = jax.lax.dot_general(
        w, xb, (((1,), (1,)), ((), ())),
        preferred_element_type=jnp.float32)
    y = y + b_ref[...]
    base = jax.nn.sigmoid(y) * msig_ref[...] + off_ref[...]
    for a in range(_A):
        slab_ref[s, :, _HW * a:_HW * (a + 1)] = \
            base[_CH * a:_CH * (a + 1), :]
    for a in range(_A):
        for d in range(2):
            yw = y[_CH * a + 2 + d:_CH * a + 3 + d, :]
            slab_ref[s, 2 + d:3 + d, _HW * a:_HW * (a + 1)] = (
                jnp.exp(yw) * float(_anch[a, d]))
    pltpu.make_async_copy(
        slab_ref.at[s], out_ref.at[:, i, :], sem.at[s]).start()

    @pl.when(i == nb - 1)
    def _():
        pltpu.make_async_copy(
            slab_ref.at[1 - s], out_ref.at[:, i - 1, :], sem.at[1 - s]).wait()
        pltpu.make_async_copy(
            slab_ref.at[s], out_ref.at[:, i, :], sem.at[s]).wait()


def kernel(x, conv_w, conv_b):
    B = x.shape[0]
    xf = jnp.transpose(x, (0, 2, 3, 1)).reshape(B, _HW, _C_IN)
    w = conv_w[:, :, 0, 0]
    b = conv_b.reshape(_C_OUT, 1)
    msig = jnp.asarray(_M_SIG).reshape(_C_OUT, 1)
    off = jnp.asarray(_OFF)

    out = pl.pallas_call(
        _yolo_kernel,
        grid=(B,),
        in_specs=[
            pl.BlockSpec((1, _HW, _C_IN), lambda i: (i, 0, 0)),
            pl.BlockSpec((_C_OUT, _C_IN), lambda i: (0, 0)),
            pl.BlockSpec((_C_OUT, 1), lambda i: (0, 0)),
            pl.BlockSpec((_C_OUT, 1), lambda i: (0, 0)),
            pl.BlockSpec((_C_OUT, _HW), lambda i: (0, 0)),
        ],
        out_specs=pl.BlockSpec(memory_space=pltpu.MemorySpace.HBM),
        out_shape=jax.ShapeDtypeStruct((_CH, B, _A * _HW), jnp.float32),
        scratch_shapes=[
            pltpu.VMEM((2, _CH, _A * _HW), jnp.float32),
            pltpu.SemaphoreType.DMA((2,)),
        ],
        compiler_params=pltpu.CompilerParams(
            dimension_semantics=("arbitrary",)),
    )(xf, w, b, msig, off)
    return jnp.transpose(out, (1, 2, 0))
